# Initial kernel scaffold; baseline (speedup 1.0000x reference)
import functools

import jax
import jax.numpy as jnp
from jax import lax
from jax.experimental import pallas as pl
from jax.experimental.pallas import tpu as pltpu
from jax.experimental.pallas import tpu_sc as plsc

N = 10000
E = 320000
D_IN = 128
D_H = 128
D_OUT = 64

NC = 2
NS = 16
NW = NC * NS
EPT = E // NW
CHUNK = 80
NCHUNK = EPT // CHUNK
RPT = N // NS
WB = 125
DEGW = 16

_mesh = plsc.VectorSubcoreMesh(core_axis_name="c", subcore_axis_name="s")


def _deg_body(dst_hbm, ones_hbm, zeros_hbm, out_hbm, idx_v, ones_v, z_v, acc_sh):
    cid = lax.axis_index("c")
    sid = lax.axis_index("s")
    wid = cid * NS + sid

    pltpu.sync_copy(zeros_hbm, z_v)
    pltpu.sync_copy(z_v, acc_sh.at[pl.ds(sid * RPT, RPT)])
    pltpu.sync_copy(ones_hbm, ones_v)
    pltpu.sync_copy(dst_hbm.at[pl.ds(wid * NCHUNK, NCHUNK)], idx_v)
    plsc.subcore_barrier()

    @pl.loop(0, NCHUNK)
    def _(c):
        pltpu.sync_copy(ones_v, acc_sh.at[idx_v.at[c]], add=True)

    plsc.subcore_barrier()
    pltpu.sync_copy(acc_sh.at[pl.ds(sid * RPT, RPT)], z_v)
    pltpu.sync_copy(z_v, out_hbm.at[pl.ds(cid * N + sid * RPT, RPT)])


def _deg_call(dst2):
    ones = jnp.ones((CHUNK, DEGW), jnp.float32)
    zeros = jnp.zeros((RPT, DEGW), jnp.float32)
    k = pl.kernel(
        _deg_body,
        out_type=jax.ShapeDtypeStruct((NC * N, DEGW), jnp.float32),
        mesh=_mesh,
        scratch_types=[
            pltpu.VMEM((NCHUNK, CHUNK), jnp.int32),
            pltpu.VMEM((CHUNK, DEGW), jnp.float32),
            pltpu.VMEM((RPT, DEGW), jnp.float32),
            pltpu.VMEM_SHARED((N, DEGW), jnp.float32),
        ],
    )
    return k(dst2, ones, zeros)


def _conv_body(h_hbm, src_hbm, dst_hbm, zeros_hbm, out_hbm,
               src_v, dst_v, rows_v, wb_v, acc_sh, sem):
    cid = lax.axis_index("c")
    sid = lax.axis_index("s")
    wid = cid * NS + sid

    pltpu.sync_copy(zeros_hbm, wb_v)
    for j in range(RPT // WB):
        pltpu.sync_copy(wb_v, acc_sh.at[pl.ds(sid * RPT + j * WB, WB)])
    pltpu.sync_copy(src_hbm.at[pl.ds(wid * NCHUNK, NCHUNK)], src_v)
    pltpu.sync_copy(dst_hbm.at[pl.ds(wid * NCHUNK, NCHUNK)], dst_v)
    plsc.subcore_barrier()

    @pl.loop(0, NCHUNK)
    def _(c):
        pltpu.async_copy(h_hbm.at[src_v.at[c]], rows_v, sem).wait()
        pltpu.sync_copy(rows_v, acc_sh.at[dst_v.at[c]], add=True)

    plsc.subcore_barrier()
    for j in range(RPT // WB):
        pltpu.sync_copy(acc_sh.at[pl.ds(sid * RPT + j * WB, WB)], wb_v)
        pltpu.sync_copy(wb_v, out_hbm.at[pl.ds(cid * N + sid * RPT + j * WB, WB)])


def _conv_call(h, src2, dst2):
    zeros = jnp.zeros((WB, D_H), jnp.float32)
    k = pl.kernel(
        _conv_body,
        out_type=jax.ShapeDtypeStruct((NC * N, D_H), jnp.float32),
        mesh=_mesh,
        scratch_types=[
            pltpu.VMEM((NCHUNK, CHUNK), jnp.int32),
            pltpu.VMEM((NCHUNK, CHUNK), jnp.int32),
            pltpu.VMEM((CHUNK, D_H), jnp.float32),
            pltpu.VMEM((WB, D_H), jnp.float32),
            pltpu.VMEM_SHARED((N, D_H), jnp.float32),
            pltpu.SemaphoreType.DMA,
        ],
    )
    return k(h, src2, dst2, zeros)


def _dinv(deg_ref):
    deg = deg_ref[0:N, 0:1] + deg_ref[N:2 * N, 0:1] + 1.0
    return lax.rsqrt(deg)


def _tc1_body(x_ref, w_ref, deg_ref, hp_ref):
    hp_ref[...] = jnp.dot(x_ref[...], w_ref[...],
                          preferred_element_type=jnp.float32) * _dinv(deg_ref)


def _tc1_call(x, W0, degparts):
    return pl.pallas_call(
        _tc1_body,
        out_shape=jax.ShapeDtypeStruct((N, D_H), jnp.float32),
    )(x, W0, degparts)


def _bn_relu(z, g, b, eps=1e-5):
    mu = jnp.mean(z, axis=0, keepdims=True)
    var = jnp.mean((z - mu) * (z - mu), axis=0, keepdims=True)
    return jnp.maximum((z - mu) * lax.rsqrt(var + eps) * g + b, 0.0)


def _tc2_body(s_ref, hp_ref, deg_ref, b_ref, g_ref, be_ref, w_ref, out_ref):
    dinv = _dinv(deg_ref)
    hp = hp_ref[...]
    z = dinv * (s_ref[0:N, :] + s_ref[N:2 * N, :] + hp) + b_ref[...]
    h = _bn_relu(z, g_ref[...], be_ref[...])
    out_ref[...] = jnp.dot(h, w_ref[...],
                           preferred_element_type=jnp.float32) * dinv


def _tc2_call(sparts, hp, degparts, b0, g0, be0, W1):
    return pl.pallas_call(
        _tc2_body,
        out_shape=jax.ShapeDtypeStruct((N, D_H), jnp.float32),
    )(sparts, hp, degparts, b0.reshape(1, -1), g0.reshape(1, -1),
      be0.reshape(1, -1), W1)


def _tc3_body(s_ref, hp_ref, deg_ref, b_ref, g_ref, be_ref,
              wl1_ref, bl1_ref, wl2_ref, bl2_ref, out_ref):
    dinv = _dinv(deg_ref)
    z = dinv * (s_ref[0:N, :] + s_ref[N:2 * N, :] + hp_ref[...]) + b_ref[...]
    h = _bn_relu(z, g_ref[...], be_ref[...])
    h = jnp.maximum(jnp.dot(h, wl1_ref[...],
                            preferred_element_type=jnp.float32) + bl1_ref[...], 0.0)
    out_ref[...] = jnp.dot(h, wl2_ref[...],
                           preferred_element_type=jnp.float32) + bl2_ref[...]


def _tc3_call(sparts, hp, degparts, b1, g1, be1, Wl1, bl1, Wl2, bl2):
    return pl.pallas_call(
        _tc3_body,
        out_shape=jax.ShapeDtypeStruct((N, D_OUT), jnp.float32),
    )(sparts, hp, degparts, b1.reshape(1, -1), g1.reshape(1, -1),
      be1.reshape(1, -1), Wl1, bl1.reshape(1, -1), Wl2, bl2.reshape(1, -1))


@jax.jit
def kernel(x, edge_index, W0, b0, W1, b1, g0, be0, g1, be1, Wl1, bl1, Wl2, bl2):
    src2 = edge_index[0].reshape(E // CHUNK, CHUNK)
    dst2 = edge_index[1].reshape(E // CHUNK, CHUNK)

    degparts = _deg_call(dst2)
    h0p = _tc1_call(x, W0, degparts)
    s0 = _conv_call(h0p, src2, dst2)
    h1p = _tc2_call(s0, h0p, degparts, b0, g0, be0, W1)
    s1 = _conv_call(h1p, src2, dst2)
    return _tc3_call(s1, h1p, degparts, b1, g1, be1, Wl1, bl1, Wl2, bl2)


# speedup vs baseline: 17.2994x; 17.2994x over previous
import jax
import jax.numpy as jnp
from jax import lax
from jax.experimental import pallas as pl
from jax.experimental.pallas import tpu as pltpu
from jax.experimental.pallas import tpu_sc as plsc

N = 10000
E = 320000
D_IN = 128
D_H = 128
D_OUT = 64

NC = 2
NS = 16
NW = NC * NS
EPT = E // NW
CHUNK = 80
NCHUNK = EPT // CHUNK
BLK = 5
NBLK = NCHUNK // BLK
NPAD = 10240
RPT = NPAD // NS
WB = 64
DEGW = 1

_mesh = plsc.VectorSubcoreMesh(core_axis_name="c", subcore_axis_name="s")


def _deg_body(dst_hbm, ones_hbm, zeros_hbm, out_hbm, idx_v, ones_v, z_v, acc_sh):
    cid = lax.axis_index("c")
    sid = lax.axis_index("s")
    wid = cid * NS + sid

    pltpu.sync_copy(zeros_hbm, z_v)
    pltpu.sync_copy(z_v, acc_sh.at[pl.ds(sid * RPT, RPT)])
    pltpu.sync_copy(ones_hbm, ones_v)
    plsc.subcore_barrier()

    @pl.loop(0, NBLK)
    def _(b):
        pltpu.sync_copy(dst_hbm.at[wid, b], idx_v)
        for j in range(BLK):
            pltpu.sync_copy(ones_v, acc_sh.at[idx_v.at[j]], add=True)

    plsc.subcore_barrier()
    pltpu.sync_copy(acc_sh.at[pl.ds(sid * RPT, RPT)], z_v)
    pltpu.sync_copy(z_v, out_hbm.at[pl.ds(cid * NPAD + sid * RPT, RPT)])


def _deg_call(dst4):
    ones = jnp.ones((CHUNK,), jnp.float32)
    zeros = jnp.zeros((RPT,), jnp.float32)
    k = pl.kernel(
        _deg_body,
        out_type=jax.ShapeDtypeStruct((NC * NPAD,), jnp.float32),
        mesh=_mesh,
        scratch_types=[
            pltpu.VMEM((BLK, CHUNK), jnp.int32),
            pltpu.VMEM((CHUNK,), jnp.float32),
            pltpu.VMEM((RPT,), jnp.float32),
            pltpu.VMEM_SHARED((NPAD,), jnp.float32),
        ],
    )
    return k(dst4, ones, zeros)


def _conv_body(h_hbm, src_hbm, dst_hbm, zeros_hbm, out_hbm,
               src_v, dst_v, rows_v, wb_v, acc_sh, sem):
    cid = lax.axis_index("c")
    sid = lax.axis_index("s")
    wid = cid * NS + sid

    pltpu.sync_copy(zeros_hbm, wb_v)
    for j in range(RPT // WB):
        pltpu.sync_copy(wb_v, acc_sh.at[pl.ds(sid * RPT + j * WB, WB)])
    plsc.subcore_barrier()

    @pl.loop(0, NBLK)
    def _(b):
        pltpu.sync_copy(src_hbm.at[wid, b], src_v)
        pltpu.sync_copy(dst_hbm.at[wid, b], dst_v)
        for j in range(BLK):
            pltpu.async_copy(h_hbm.at[src_v.at[j]], rows_v, sem).wait()
            pltpu.sync_copy(rows_v, acc_sh.at[dst_v.at[j]], add=True)

    plsc.subcore_barrier()
    for j in range(RPT // WB):
        pltpu.sync_copy(acc_sh.at[pl.ds(sid * RPT + j * WB, WB)], wb_v)
        pltpu.sync_copy(
            wb_v, out_hbm.at[pl.ds(cid * NPAD + sid * RPT + j * WB, WB)])


def _conv_call(h, src4, dst4):
    zeros = jnp.zeros((WB, D_H), jnp.float32)
    k = pl.kernel(
        _conv_body,
        out_type=jax.ShapeDtypeStruct((NC * NPAD, D_H), jnp.float32),
        mesh=_mesh,
        scratch_types=[
            pltpu.VMEM((BLK, CHUNK), jnp.int32),
            pltpu.VMEM((BLK, CHUNK), jnp.int32),
            pltpu.VMEM((CHUNK, D_H), jnp.float32),
            pltpu.VMEM((WB, D_H), jnp.float32),
            pltpu.VMEM_SHARED((NPAD, D_H), jnp.float32),
            pltpu.SemaphoreType.DMA,
        ],
    )
    return k(h, src4, dst4, zeros)


def _dinv(deg_ref):
    deg = deg_ref[0:N, 0:1] + deg_ref[NPAD:NPAD + N, 0:1] + 1.0
    return lax.rsqrt(deg)


def _tc1_body(x_ref, w_ref, deg_ref, hp_ref):
    hp_ref[...] = jnp.dot(x_ref[...], w_ref[...],
                          preferred_element_type=jnp.float32) * _dinv(deg_ref)


def _tc1_call(x, W0, degparts):
    return pl.pallas_call(
        _tc1_body,
        out_shape=jax.ShapeDtypeStruct((N, D_H), jnp.float32),
    )(x, W0, degparts)


def _bn_relu(z, g, b, eps=1e-5):
    mu = jnp.mean(z, axis=0, keepdims=True)
    var = jnp.mean((z - mu) * (z - mu), axis=0, keepdims=True)
    return jnp.maximum((z - mu) * lax.rsqrt(var + eps) * g + b, 0.0)


def _tc2_body(s_ref, hp_ref, deg_ref, b_ref, g_ref, be_ref, w_ref, out_ref):
    dinv = _dinv(deg_ref)
    hp = hp_ref[...]
    z = dinv * (s_ref[0:N, :] + s_ref[NPAD:NPAD + N, :] + hp) + b_ref[...]
    h = _bn_relu(z, g_ref[...], be_ref[...])
    out_ref[...] = jnp.dot(h, w_ref[...],
                           preferred_element_type=jnp.float32) * dinv


def _tc2_call(sparts, hp, degparts, b0, g0, be0, W1):
    return pl.pallas_call(
        _tc2_body,
        out_shape=jax.ShapeDtypeStruct((N, D_H), jnp.float32),
    )(sparts, hp, degparts, b0.reshape(1, -1), g0.reshape(1, -1),
      be0.reshape(1, -1), W1)


def _tc3_body(s_ref, hp_ref, deg_ref, b_ref, g_ref, be_ref,
              wl1_ref, bl1_ref, wl2_ref, bl2_ref, out_ref):
    dinv = _dinv(deg_ref)
    z = dinv * (s_ref[0:N, :] + s_ref[NPAD:NPAD + N, :] + hp_ref[...]) + b_ref[...]
    h = _bn_relu(z, g_ref[...], be_ref[...])
    h = jnp.maximum(jnp.dot(h, wl1_ref[...],
                            preferred_element_type=jnp.float32) + bl1_ref[...], 0.0)
    out_ref[...] = jnp.dot(h, wl2_ref[...],
                           preferred_element_type=jnp.float32) + bl2_ref[...]


def _tc3_call(sparts, hp, degparts, b1, g1, be1, Wl1, bl1, Wl2, bl2):
    return pl.pallas_call(
        _tc3_body,
        out_shape=jax.ShapeDtypeStruct((N, D_OUT), jnp.float32),
    )(sparts, hp, degparts, b1.reshape(1, -1), g1.reshape(1, -1),
      be1.reshape(1, -1), Wl1, bl1.reshape(1, -1), Wl2, bl2.reshape(1, -1))


@jax.jit
def kernel(x, edge_index, W0, b0, W1, b1, g0, be0, g1, be1, Wl1, bl1, Wl2, bl2):
    src4 = edge_index[0].reshape(NW, NBLK, BLK, CHUNK)
    dst4 = edge_index[1].reshape(NW, NBLK, BLK, CHUNK)

    degparts = _deg_call(dst4).reshape(NC * NPAD, 1)
    h0p = _tc1_call(x, W0, degparts)
    s0 = _conv_call(h0p, src4, dst4)
    h1p = _tc2_call(s0, h0p, degparts, b0, g0, be0, W1)
    s1 = _conv_call(h1p, src4, dst4)
    return _tc3_call(s1, h1p, degparts, b1, g1, be1, Wl1, bl1, Wl2, bl2)


# speedup vs baseline: 23.7589x; 1.3734x over previous
import jax
import jax.numpy as jnp
from jax import lax
from jax.experimental import pallas as pl
from jax.experimental.pallas import tpu as pltpu
from jax.experimental.pallas import tpu_sc as plsc

N = 10000
E = 320000
D_IN = 128
D_H = 128
D_OUT = 64

NC = 2
NS = 16
NW = NC * NS
EPT = E // NW
CHUNK = 100
NCHUNK = EPT // CHUNK
BLK = 10
NBLK = NCHUNK // BLK
NPAD = 10240
RPT = NPAD // NS
WB = 32
DEGW = 1

_mesh = plsc.VectorSubcoreMesh(core_axis_name="c", subcore_axis_name="s")


def _deg_body(dst_hbm, ones_hbm, zeros_hbm, out_hbm, idx_v, ones_v, z_v, acc_sh):
    cid = lax.axis_index("c")
    sid = lax.axis_index("s")
    wid = cid * NS + sid

    pltpu.sync_copy(zeros_hbm, z_v)
    pltpu.sync_copy(z_v, acc_sh.at[pl.ds(sid * RPT, RPT)])
    pltpu.sync_copy(ones_hbm, ones_v)
    plsc.subcore_barrier()

    @pl.loop(0, NBLK)
    def _(b):
        pltpu.sync_copy(dst_hbm.at[wid, b], idx_v)
        for j in range(BLK):
            pltpu.sync_copy(ones_v, acc_sh.at[idx_v.at[j]], add=True)

    plsc.subcore_barrier()
    pltpu.sync_copy(acc_sh.at[pl.ds(sid * RPT, RPT)], z_v)
    pltpu.sync_copy(z_v, out_hbm.at[pl.ds(cid * NPAD + sid * RPT, RPT)])


def _deg_call(dst4):
    ones = jnp.ones((CHUNK,), jnp.float32)
    zeros = jnp.zeros((RPT,), jnp.float32)
    k = pl.kernel(
        _deg_body,
        out_type=jax.ShapeDtypeStruct((NC * NPAD,), jnp.float32),
        mesh=_mesh,
        scratch_types=[
            pltpu.VMEM((BLK, CHUNK), jnp.int32),
            pltpu.VMEM((CHUNK,), jnp.float32),
            pltpu.VMEM((RPT,), jnp.float32),
            pltpu.VMEM_SHARED((NPAD,), jnp.float32),
        ],
    )
    return k(dst4, ones, zeros)


def _conv_body(h_hbm, src_hbm, dst_hbm, zeros_hbm, out_hbm,
               src_v, dst_v, rows_a, rows_b, wb_v, acc_sh, sem_a, sem_b):
    cid = lax.axis_index("c")
    sid = lax.axis_index("s")
    wid = cid * NS + sid

    pltpu.sync_copy(zeros_hbm, wb_v)
    for j in range(RPT // WB):
        pltpu.sync_copy(wb_v, acc_sh.at[pl.ds(sid * RPT + j * WB, WB)])
    plsc.subcore_barrier()

    rows = (rows_a, rows_b)
    sems = (sem_a, sem_b)

    @pl.loop(0, NBLK)
    def _(b):
        pltpu.sync_copy(src_hbm.at[wid, b], src_v)
        pltpu.sync_copy(dst_hbm.at[wid, b], dst_v)
        cp = pltpu.async_copy(h_hbm.at[src_v.at[0]], rows[0], sems[0])
        for j in range(BLK):
            cp.wait()
            if j + 1 < BLK:
                cp = pltpu.async_copy(
                    h_hbm.at[src_v.at[j + 1]], rows[(j + 1) % 2],
                    sems[(j + 1) % 2])
            pltpu.sync_copy(rows[j % 2], acc_sh.at[dst_v.at[j]], add=True)

    plsc.subcore_barrier()
    for j in range(RPT // WB):
        pltpu.sync_copy(acc_sh.at[pl.ds(sid * RPT + j * WB, WB)], wb_v)
        pltpu.sync_copy(
            wb_v, out_hbm.at[pl.ds(cid * NPAD + sid * RPT + j * WB, WB)])


def _conv_call(h, src4, dst4):
    zeros = jnp.zeros((WB, D_H), jnp.float32)
    k = pl.kernel(
        _conv_body,
        out_type=jax.ShapeDtypeStruct((NC * NPAD, D_H), jnp.float32),
        mesh=_mesh,
        scratch_types=[
            pltpu.VMEM((BLK, CHUNK), jnp.int32),
            pltpu.VMEM((BLK, CHUNK), jnp.int32),
            pltpu.VMEM((CHUNK, D_H), jnp.float32),
            pltpu.VMEM((CHUNK, D_H), jnp.float32),
            pltpu.VMEM((WB, D_H), jnp.float32),
            pltpu.VMEM_SHARED((NPAD, D_H), jnp.float32),
            pltpu.SemaphoreType.DMA,
            pltpu.SemaphoreType.DMA,
        ],
    )
    return k(h, src4, dst4, zeros)


def _dinv(deg_ref):
    deg = deg_ref[0:N, 0:1] + deg_ref[NPAD:NPAD + N, 0:1] + 1.0
    return lax.rsqrt(deg)


def _tc1_body(x_ref, w_ref, deg_ref, hp_ref):
    hp_ref[...] = jnp.dot(x_ref[...], w_ref[...],
                          preferred_element_type=jnp.float32) * _dinv(deg_ref)


def _tc1_call(x, W0, degparts):
    return pl.pallas_call(
        _tc1_body,
        out_shape=jax.ShapeDtypeStruct((N, D_H), jnp.float32),
    )(x, W0, degparts)


def _bn_relu(z, g, b, eps=1e-5):
    mu = jnp.mean(z, axis=0, keepdims=True)
    var = jnp.mean((z - mu) * (z - mu), axis=0, keepdims=True)
    return jnp.maximum((z - mu) * lax.rsqrt(var + eps) * g + b, 0.0)


def _tc2_body(s_ref, hp_ref, deg_ref, b_ref, g_ref, be_ref, w_ref, out_ref):
    dinv = _dinv(deg_ref)
    hp = hp_ref[...]
    z = dinv * (s_ref[0:N, :] + s_ref[NPAD:NPAD + N, :] + hp) + b_ref[...]
    h = _bn_relu(z, g_ref[...], be_ref[...])
    out_ref[...] = jnp.dot(h, w_ref[...],
                           preferred_element_type=jnp.float32) * dinv


def _tc2_call(sparts, hp, degparts, b0, g0, be0, W1):
    return pl.pallas_call(
        _tc2_body,
        out_shape=jax.ShapeDtypeStruct((N, D_H), jnp.float32),
    )(sparts, hp, degparts, b0.reshape(1, -1), g0.reshape(1, -1),
      be0.reshape(1, -1), W1)


def _tc3_body(s_ref, hp_ref, deg_ref, b_ref, g_ref, be_ref,
              wl1_ref, bl1_ref, wl2_ref, bl2_ref, out_ref):
    dinv = _dinv(deg_ref)
    z = dinv * (s_ref[0:N, :] + s_ref[NPAD:NPAD + N, :] + hp_ref[...]) + b_ref[...]
    h = _bn_relu(z, g_ref[...], be_ref[...])
    h = jnp.maximum(jnp.dot(h, wl1_ref[...],
                            preferred_element_type=jnp.float32) + bl1_ref[...], 0.0)
    out_ref[...] = jnp.dot(h, wl2_ref[...],
                           preferred_element_type=jnp.float32) + bl2_ref[...]


def _tc3_call(sparts, hp, degparts, b1, g1, be1, Wl1, bl1, Wl2, bl2):
    return pl.pallas_call(
        _tc3_body,
        out_shape=jax.ShapeDtypeStruct((N, D_OUT), jnp.float32),
    )(sparts, hp, degparts, b1.reshape(1, -1), g1.reshape(1, -1),
      be1.reshape(1, -1), Wl1, bl1.reshape(1, -1), Wl2, bl2.reshape(1, -1))


@jax.jit
def kernel(x, edge_index, W0, b0, W1, b1, g0, be0, g1, be1, Wl1, bl1, Wl2, bl2):
    src4 = edge_index[0].reshape(NW, NBLK, BLK, CHUNK)
    dst4 = edge_index[1].reshape(NW, NBLK, BLK, CHUNK)

    degparts = _deg_call(dst4).reshape(NC * NPAD, 1)
    h0p = _tc1_call(x, W0, degparts)
    s0 = _conv_call(h0p, src4, dst4)
    h1p = _tc2_call(s0, h0p, degparts, b0, g0, be0, W1)
    s1 = _conv_call(h1p, src4, dst4)
    return _tc3_call(s1, h1p, degparts, b1, g1, be1, Wl1, bl1, Wl2, bl2)
